# baseline (speedup 1.0000x reference)
import functools

import jax
import jax.numpy as jnp
from jax import lax
from jax.experimental import pallas as pl
from jax.experimental.pallas import tpu as pltpu
from jax.experimental.pallas import tpu_sc as plsc

B = 4096
L = 200
D = 64
LP = 208
CH = LP // 16
NPAD = LP - L
NC = 2
NS = 16
NW = NC * NS
RPW = B // NW
NV = D // 16
NBUF = 4


def _sc_call(inputs2, emb_table, wtab):
    mesh = plsc.VectorSubcoreMesh(core_axis_name="c", subcore_axis_name="s")

    @functools.partial(
        pl.kernel,
        out_type=jax.ShapeDtypeStruct((B, D), jnp.float32),
        mesh=mesh,
        scratch_types=[
            pltpu.VMEM((RPW * LP,), jnp.int32),
            pltpu.VMEM((NBUF, LP, D), jnp.float32),
            pltpu.VMEM((RPW * LP,), jnp.float32),
            pltpu.VMEM((RPW, D), jnp.float32),
            [pltpu.SemaphoreType.DMA] * NBUF,
            pltpu.SemaphoreType.DMA,
        ],
        compiler_params=pltpu.CompilerParams(use_tc_tiling_on_sc=False),
    )
    def k(inputs_hbm, emb_hbm, w_hbm, out_hbm,
          idx_v, rows_v, w_all, res_v, sems, wsem):
        wid = lax.axis_index("s") * NC + lax.axis_index("c")
        base = wid * RPW * LP
        pltpu.sync_copy(inputs_hbm.at[pl.ds(base, RPW * LP)], idx_v)
        wcp = pltpu.async_copy(w_hbm.at[idx_v], w_all, wsem)

        def issue(row, s):
            pltpu.async_copy(
                emb_hbm.at[idx_v.at[pl.ds(row * LP, LP)]],
                rows_v.at[s], sems[s])

        def drain(s):
            pltpu.make_async_copy(
                emb_hbm.at[pl.ds(0, LP)], rows_v.at[s], sems[s]).wait()

        def accum(acc, w16, l0, s, nlanes=16):
            acc = list(acc)
            for i in range(nlanes):
                wi = w16[i]
                for kv in range(NV):
                    acc[kv] = acc[kv] + wi * rows_v[
                        s, l0 + i, pl.ds(kv * 16, 16)]
            return tuple(acc)

        def compute(row, s):
            acc = tuple(jnp.zeros((16,), jnp.float32) for _ in range(NV))

            def c_body(c, acc):
                l0 = c * 16
                w16 = w_all[pl.ds(row * LP + l0, 16)]
                return accum(acc, w16, l0, s)

            acc = lax.fori_loop(0, CH - 1, c_body, acc)
            l0 = (CH - 1) * 16
            w16 = w_all[pl.ds(row * LP + l0, 16)]
            acc = accum(acc, w16, l0, s, nlanes=16 - NPAD)
            for kv in range(NV):
                res_v[row, pl.ds(kv * 16, 16)] = acc[kv]

        for s in range(NBUF - 1):
            issue(s, s)
        wcp.wait()

        def g_body(g, carry):
            for s in range(NBUF):
                row = g * NBUF + s

                @pl.when(row + NBUF - 1 < RPW)
                def _():
                    issue(row + NBUF - 1, (s + NBUF - 1) % NBUF)

                drain(s)
                compute(row, s)
            return carry

        lax.fori_loop(0, RPW // NBUF, g_body, 0)
        pltpu.sync_copy(res_v, out_hbm.at[pl.ds(wid * RPW, RPW)])

    return k(inputs2, emb_table, wtab)


def kernel(inputs, emb_table, weight_table):
    wtab = weight_table[:, 0].at[0].set(0.0)
    pad = (jnp.arange(B, dtype=jnp.int32)[:, None] * NPAD
           + jnp.arange(NPAD, dtype=jnp.int32)[None, :] + 1)
    inputs2 = jnp.concatenate([inputs, pad], axis=1).reshape(-1)
    return _sc_call(inputs2, emb_table, wtab)


# speedup vs baseline: 1.0065x; 1.0065x over previous
import functools

import jax
import jax.numpy as jnp
from jax import lax
from jax.experimental import pallas as pl
from jax.experimental.pallas import tpu as pltpu
from jax.experimental.pallas import tpu_sc as plsc

B = 4096
L = 200
D = 64
LP = 208
CH = LP // 16
NPAD = LP - L
NC = 2
NS = 16
NW = NC * NS
RPW = B // NW
NV = D // 16
NBUF = 4


def _sc_call(inputs2, emb_table, wtab):
    mesh = plsc.VectorSubcoreMesh(core_axis_name="c", subcore_axis_name="s")

    @functools.partial(
        pl.kernel,
        out_type=jax.ShapeDtypeStruct((B, D), jnp.float32),
        mesh=mesh,
        scratch_types=[
            pltpu.VMEM((RPW * LP,), jnp.int32),
            pltpu.VMEM((NBUF, LP, D), jnp.float32),
            pltpu.VMEM((RPW * LP,), jnp.float32),
            pltpu.VMEM((RPW, D), jnp.float32),
            [pltpu.SemaphoreType.DMA] * NBUF,
            pltpu.SemaphoreType.DMA,
        ],
        compiler_params=pltpu.CompilerParams(use_tc_tiling_on_sc=False),
    )
    def k(inputs_hbm, emb_hbm, w_hbm, out_hbm,
          idx_v, rows_v, w_all, res_v, sems, wsem):
        wid = lax.axis_index("s") * NC + lax.axis_index("c")
        base = wid * RPW * LP
        pltpu.sync_copy(inputs_hbm.at[pl.ds(base, RPW * LP)], idx_v)
        wcp = pltpu.async_copy(w_hbm.at[idx_v], w_all, wsem)

        def issue(row, s):
            pltpu.async_copy(
                emb_hbm.at[idx_v.at[pl.ds(row * LP, LP)]],
                rows_v.at[s], sems[s])

        def drain(s):
            pltpu.make_async_copy(
                emb_hbm.at[pl.ds(0, LP)], rows_v.at[s], sems[s]).wait()

        def accum(acc, w16, l0, s, nlanes=16):
            acc = list(acc)
            for i in range(nlanes):
                wi = w16[i]
                for kv in range(NV):
                    acc[kv] = acc[kv] + wi * rows_v[
                        s, l0 + i, pl.ds(kv * 16, 16)]
            return tuple(acc)

        def compute(row, s):
            acc = tuple(jnp.zeros((16,), jnp.float32) for _ in range(NV))

            def c_body(c, acc):
                l0 = c * 16
                w16 = w_all[pl.ds(row * LP + l0, 16)]
                ix16 = idx_v[pl.ds(row * LP + l0, 16)]
                w16 = jnp.where(ix16 == 0, 0.0, w16)
                return accum(acc, w16, l0, s)

            acc = lax.fori_loop(0, CH - 1, c_body, acc)
            l0 = (CH - 1) * 16
            w16 = w_all[pl.ds(row * LP + l0, 16)]
            ix16 = idx_v[pl.ds(row * LP + l0, 16)]
            w16 = jnp.where(ix16 == 0, 0.0, w16)
            acc = accum(acc, w16, l0, s, nlanes=16 - NPAD)
            for kv in range(NV):
                res_v[row, pl.ds(kv * 16, 16)] = acc[kv]

        for s in range(NBUF - 1):
            issue(s, s)
        wcp.wait()

        def g_body(g, carry):
            for s in range(NBUF):
                row = g * NBUF + s

                @pl.when(row + NBUF - 1 < RPW)
                def _():
                    issue(row + NBUF - 1, (s + NBUF - 1) % NBUF)

                drain(s)
                compute(row, s)
            return carry

        lax.fori_loop(0, RPW // NBUF, g_body, 0)
        pltpu.sync_copy(res_v, out_hbm.at[pl.ds(wid * RPW, RPW)])

    return k(inputs2, emb_table, wtab)


def kernel(inputs, emb_table, weight_table):
    wtab = weight_table[:, 0]
    pad = (jnp.arange(B, dtype=jnp.int32)[:, None] * NPAD
           + jnp.arange(NPAD, dtype=jnp.int32)[None, :] + 1)
    inputs2 = jnp.concatenate([inputs, pad], axis=1).reshape(-1)
    return _sc_call(inputs2, emb_table, wtab)
